# Initial kernel scaffold; baseline (speedup 1.0000x reference)
import jax
import jax.numpy as jnp
from jax.experimental import pallas as pl


def kernel(input_ids, token_type_ids, visual_embeds, visual_token_type_ids, word_emb, pos_emb, tok_type_emb, vis_tok_type_emb, vis_pos_emb, vproj_w, vproj_b, ln_gamma, ln_beta):
    raise NotImplementedError("write your pallas kernel here")


# speedup vs baseline: 1.4442x; 1.4442x over previous
import functools

import jax
import jax.numpy as jnp
from jax import lax
from jax.experimental import pallas as pl
from jax.experimental.pallas import tpu as pltpu
from jax.experimental.pallas import tpu_sc as plsc

_EPS = 1e-12

_NC = 2
_NS = 16
_NW = _NC * _NS


def _sc_gather(table, idx):
    BS = idx.shape[0]
    H = table.shape[1]
    b_per_w = BS // _NW
    CH = 64
    n_ch = b_per_w // CH

    mesh = plsc.VectorSubcoreMesh(core_axis_name="c", subcore_axis_name="s")

    @functools.partial(
        pl.kernel,
        mesh=mesh,
        out_type=jax.ShapeDtypeStruct((BS, H), jnp.float32),
        scratch_types=[
            pltpu.VMEM((CH,), jnp.int32),
            pltpu.VMEM((CH, H), jnp.float32),
            pltpu.SemaphoreType.DMA,
        ],
    )
    def k(idx_hbm, table_hbm, out_hbm, idx_v, rows_v, sem):
        wid = lax.axis_index("s") * _NC + lax.axis_index("c")
        base = wid * b_per_w

        def body(i, carry):
            off = base + i * CH
            pltpu.sync_copy(idx_hbm.at[pl.ds(off, CH)], idx_v)
            pltpu.async_copy(table_hbm.at[idx_v], rows_v, sem).wait()
            pltpu.sync_copy(rows_v, out_hbm.at[pl.ds(off, CH)])
            return carry

        lax.fori_loop(0, n_ch, body, 0)

    return k(idx, table)


def _tc_fused(gathered, tt_ids, ve, vtt_ids, pos_s, tte, vtte, vpos0,
              w, b2, gam, bet, BB):
    B, S, H = gathered.shape
    V, VD = ve.shape[1], ve.shape[2]
    grid = (B // BB,)
    ttf = tt_ids.astype(jnp.float32).reshape(B, S, 1)
    vttf = vtt_ids.astype(jnp.float32).reshape(B, V, 1)

    def body(g_ref, tt_ref, ve_ref, vtt_ref, pos_ref, tte_ref, vtte_ref,
             vpos_ref, w_ref, b_ref, gam_ref, bet_ref, o_ref):
        g = gam_ref[0, :]
        be = bet_ref[0, :]

        def ln(x):
            mu = jnp.mean(x, axis=-1, keepdims=True)
            xc = x - mu
            var = jnp.mean(xc * xc, axis=-1, keepdims=True)
            return xc * lax.rsqrt(var + _EPS) * g[None, :] + be[None, :]

        t0 = tte_ref[0, :]
        t1 = tte_ref[1, :]
        v0 = vtte_ref[0, :] + vpos_ref[0, :] + b_ref[0, :]
        v1 = vtte_ref[1, :] + vpos_ref[0, :] + b_ref[0, :]
        pos = pos_ref[...]
        for bb in range(BB):
            tsel = tt_ref[bb]
            text = (g_ref[bb] + pos + t0[None, :]
                    + tsel * (t1 - t0)[None, :])
            o_ref[bb, :S, :] = ln(text)
            vm = lax.dot_general(ve_ref[bb], w_ref[...],
                                 (((1,), (1,)), ((), ())),
                                 preferred_element_type=jnp.float32)
            vsel = vtt_ref[bb]
            vis = vm + v0[None, :] + vsel * (v1 - v0)[None, :]
            o_ref[bb, S:, :] = ln(vis)

    return pl.pallas_call(
        body,
        grid=grid,
        in_specs=[
            pl.BlockSpec((BB, S, H), lambda i: (i, 0, 0)),
            pl.BlockSpec((BB, S, 1), lambda i: (i, 0, 0)),
            pl.BlockSpec((BB, V, VD), lambda i: (i, 0, 0)),
            pl.BlockSpec((BB, V, 1), lambda i: (i, 0, 0)),
            pl.BlockSpec((S, H), lambda i: (0, 0)),
            pl.BlockSpec((2, H), lambda i: (0, 0)),
            pl.BlockSpec((2, H), lambda i: (0, 0)),
            pl.BlockSpec((1, H), lambda i: (0, 0)),
            pl.BlockSpec((H, VD), lambda i: (0, 0)),
            pl.BlockSpec((1, H), lambda i: (0, 0)),
            pl.BlockSpec((1, H), lambda i: (0, 0)),
            pl.BlockSpec((1, H), lambda i: (0, 0)),
        ],
        out_specs=pl.BlockSpec((BB, S + V, H), lambda i: (i, 0, 0)),
        out_shape=jax.ShapeDtypeStruct((B, S + V, H), jnp.float32),
    )(gathered, ttf, ve, vttf, pos_s, tte, vtte, vpos0, w, b2, gam, bet)


def kernel(input_ids, token_type_ids, visual_embeds, visual_token_type_ids,
           word_emb, pos_emb, tok_type_emb, vis_tok_type_emb, vis_pos_emb,
           vproj_w, vproj_b, ln_gamma, ln_beta):
    B, S = input_ids.shape
    H = word_emb.shape[1]
    gathered = _sc_gather(word_emb, input_ids.reshape(-1)).reshape(B, S, H)
    return _tc_fused(
        gathered, token_type_ids, visual_embeds, visual_token_type_ids,
        pos_emb[:S], tok_type_emb, vis_tok_type_emb, vis_pos_emb[0:1],
        vproj_w, vproj_b.reshape(1, H), ln_gamma.reshape(1, H),
        ln_beta.reshape(1, H), BB=4)
